# baseline (speedup 1.0000x reference)
import functools

import jax
import jax.numpy as jnp
from jax import lax
from jax.experimental import pallas as pl
from jax.experimental.pallas import tpu as pltpu
from jax.experimental.pallas import tpu_sc as plsc

B, S, D = 16384, 6, 512
NC, NS = 2, 16
NW = NC * NS
BPW = B // NW


@functools.partial(
    pl.kernel,
    out_type=jax.ShapeDtypeStruct((B, S, D), jnp.float32),
    mesh=plsc.VectorSubcoreMesh(core_axis_name="c", subcore_axis_name="s"),
    scratch_types=[
        pltpu.VMEM((16,), jnp.int32),
        pltpu.SemaphoreType.DMA,
    ],
    compiler_params=pltpu.CompilerParams(
        use_tc_tiling_on_sc=True, needs_layout_passes=False),
)
def _shuffle_slabs(x_hbm, tab_hbm, out_hbm, tab_v, sem):
    wid = lax.axis_index("s") * NC + lax.axis_index("c")
    bw = wid * BPW

    pltpu.sync_copy(tab_hbm, tab_v)
    tab_vec = tab_v[...]
    iota = lax.broadcasted_iota(jnp.int32, (16,), 0)
    handles = []
    for i in range(S):
        pi = jnp.max(jnp.where(iota == i, tab_vec, 0))
        handles.append(pltpu.async_copy(
            x_hbm.at[pl.ds(bw, BPW), pl.ds(pi, 1), :],
            out_hbm.at[pl.ds(bw, BPW), pl.ds(i, 1), :],
            sem))
    for h in handles:
        h.wait()


def kernel(x, index):
    tab16 = jnp.zeros((16,), jnp.int32).at[:S].set(index.astype(jnp.int32))
    return _shuffle_slabs(x, tab16)


# speedup vs baseline: 13.3402x; 13.3402x over previous
import functools

import jax
import jax.numpy as jnp
from jax import lax
from jax.experimental import pallas as pl
from jax.experimental.pallas import tpu as pltpu
from jax.experimental.pallas import tpu_sc as plsc

B, S, D = 16384, 6, 512
NC, NS = 2, 16
NW = NC * NS
BPW = B // NW
CB = 64
NCHB = BPW // CB


@functools.partial(
    pl.kernel,
    out_type=jax.ShapeDtypeStruct((B, S, D), jnp.float32),
    mesh=plsc.VectorSubcoreMesh(core_axis_name="c", subcore_axis_name="s"),
    scratch_types=[
        pltpu.VMEM((16,), jnp.int32),
        pltpu.VMEM((CB, 1, D), jnp.float32),
        pltpu.VMEM((CB, 1, D), jnp.float32),
        pltpu.SemaphoreType.DMA,
        pltpu.SemaphoreType.DMA,
        pltpu.SemaphoreType.DMA,
        pltpu.SemaphoreType.DMA,
    ],
    compiler_params=pltpu.CompilerParams(
        use_tc_tiling_on_sc=True, needs_layout_passes=False),
)
def _shuffle_slabs(x_hbm, tab_hbm, out_hbm, tab_v, buf0, buf1,
                   g0, g1, w0, w1):
    wid = lax.axis_index("s") * NC + lax.axis_index("c")
    b0 = wid * BPW

    pltpu.sync_copy(tab_hbm, tab_v)
    tab_vec = tab_v[...]
    iota = lax.broadcasted_iota(jnp.int32, (16,), 0)
    pis = [jnp.max(jnp.where(iota == i, tab_vec, 0)) for i in range(S)]

    tasks = [(i, c) for i in range(S) for c in range(NCHB)]
    ntask = len(tasks)
    buf = (buf0, buf1)
    gsem = (g0, g1)
    wsem = (w0, w1)

    def gather(t):
        i, c = tasks[t]
        return pltpu.async_copy(
            x_hbm.at[pl.ds(b0 + c * CB, CB), pl.ds(pis[i], 1), :],
            buf[t % 2], gsem[t % 2])

    def write(t):
        i, c = tasks[t]
        return pltpu.async_copy(
            buf[t % 2],
            out_hbm.at[pl.ds(b0 + c * CB, CB), pl.ds(i, 1), :],
            wsem[t % 2])

    gh = [None, None]
    wh = [None, None]
    gh[0] = gather(0)
    for t in range(ntask):
        b = t % 2
        gh[b].wait()
        wh[b] = write(t)
        if t + 1 < ntask:
            nb = (t + 1) % 2
            if wh[nb] is not None:
                wh[nb].wait()
            gh[nb] = gather(t + 1)
    wh[0].wait()
    wh[1].wait()


def kernel(x, index):
    tab16 = jnp.zeros((16,), jnp.int32).at[:S].set(index.astype(jnp.int32))
    return _shuffle_slabs(x, tab16)


# speedup vs baseline: 13.3402x; 1.0000x over previous
import functools

import jax
import jax.numpy as jnp
from jax import lax
from jax.experimental import pallas as pl
from jax.experimental.pallas import tpu as pltpu
from jax.experimental.pallas import tpu_sc as plsc

B, S, D = 16384, 6, 512
NC, NS = 2, 16
NW = NC * NS
BPW = B // NW
CB = 64
NCHB = BPW // CB


@functools.partial(
    pl.kernel,
    out_type=jax.ShapeDtypeStruct((B, S, D), jnp.float32),
    mesh=plsc.VectorSubcoreMesh(core_axis_name="c", subcore_axis_name="s"),
    scratch_types=[
        pltpu.VMEM((16,), jnp.int32),
        pltpu.VMEM((CB, 1, D), jnp.float32),
        pltpu.VMEM((CB, 1, D), jnp.float32),
        pltpu.VMEM((CB, 1, D), jnp.float32),
        pltpu.SemaphoreType.DMA,
        pltpu.SemaphoreType.DMA,
        pltpu.SemaphoreType.DMA,
        pltpu.SemaphoreType.DMA,
        pltpu.SemaphoreType.DMA,
        pltpu.SemaphoreType.DMA,
    ],
    compiler_params=pltpu.CompilerParams(
        use_tc_tiling_on_sc=True, needs_layout_passes=False),
)
def _shuffle_slabs(x_hbm, tab_hbm, out_hbm, tab_v, buf0, buf1, buf2,
                   g0, g1, g2, w0, w1, w2):
    wid = lax.axis_index("s") * NC + lax.axis_index("c")
    b0 = wid * BPW

    pltpu.sync_copy(tab_hbm, tab_v)
    tab_vec = tab_v[...]
    iota = lax.broadcasted_iota(jnp.int32, (16,), 0)
    pis = [jnp.max(jnp.where(iota == i, tab_vec, 0)) for i in range(S)]

    tasks = [(i, c) for i in range(S) for c in range(NCHB)]
    ntask = len(tasks)
    NB = 3
    buf = (buf0, buf1, buf2)
    gsem = (g0, g1, g2)
    wsem = (w0, w1, w2)

    def gather(t):
        i, c = tasks[t]
        return pltpu.async_copy(
            x_hbm.at[pl.ds(b0 + c * CB, CB), pl.ds(pis[i], 1), :],
            buf[t % NB], gsem[t % NB])

    def write(t):
        i, c = tasks[t]
        return pltpu.async_copy(
            buf[t % NB],
            out_hbm.at[pl.ds(b0 + c * CB, CB), pl.ds(i, 1), :],
            wsem[t % NB])

    gh = [None] * NB
    wh = [None] * NB
    for t in range(min(NB - 1, ntask)):
        gh[t % NB] = gather(t)
    for t in range(ntask):
        b = t % NB
        if t + NB - 1 < ntask:
            nb = (t + NB - 1) % NB
            if wh[nb] is not None:
                wh[nb].wait()
            gh[nb] = gather(t + NB - 1)
        gh[b].wait()
        wh[b] = write(t)
    for b in range(NB):
        if wh[b] is not None:
            wh[b].wait()


def kernel(x, index):
    tab16 = jnp.zeros((16,), jnp.int32).at[:S].set(index.astype(jnp.int32))
    return _shuffle_slabs(x, tab16)
